# baseline (speedup 1.0000x reference)
import dataclasses
import functools

import jax
import jax.numpy as jnp
from jax import lax
from jax.experimental import pallas as pl
from jax.experimental.pallas import tpu as pltpu
from jax.experimental.pallas import tpu_sc as plsc

INPUT_SIZE = 65536
SIZE = 128
STRIDE = 16
NPZ = 64
K = 8
NUM_ZONES = (INPUT_SIZE - (SIZE - 1)) // STRIDE

NW = 32
ZPW = 128
XSLAB = ZPW * STRIDE + SIZE
WZ = NPZ * SIZE
NEG_INF = float('-inf')


def _round_bf16(v):
    u = plsc.bitcast(v, jnp.int32)
    r = u + jnp.int32(0x7FFF) + ((u >> 16) & jnp.int32(1))
    return plsc.bitcast(r & jnp.int32(-65536), jnp.float32)


def _tec_body(x_hbm, w_hbm, o_hbm, xbuf, wbuf, obuf, sx, s0, s1, so):
    wid = lax.axis_index("s") * 2 + lax.axis_index("c")
    z0 = jnp.minimum(wid * ZPW, NUM_ZONES - ZPW)

    cpx = pltpu.make_async_copy(
        x_hbm.at[pl.ds(z0 * STRIDE, XSLAB)], xbuf, sx)
    cpx.start()
    cpx.wait()

    @pl.loop(0, XSLAB, step=16)
    def _(c):
        xbuf[pl.ds(c, 16)] = _round_bf16(xbuf[pl.ds(c, 16)])

    lane = lax.broadcasted_iota(jnp.int32, (16,), 0)
    gidx = [lane * SIZE + 16 * g * SIZE for g in range(NPZ // 16)]

    def fire(k, boff, sem):
        cp = pltpu.make_async_copy(
            w_hbm.at[z0 + k], wbuf.at[pl.ds(boff, WZ)], sem)
        cp.start()

    def wait(boff, sem):
        pltpu.make_async_copy(
            w_hbm.at[0], wbuf.at[pl.ds(boff, WZ)], sem).wait()

    def compute_zone(k, boff):
        xvec = jnp.full((16,), k * STRIDE, jnp.int32)

        def sbody(s, accs):
            ws = plsc.load_gather(xbuf, [xvec + s])
            new = []
            for g in range(NPZ // 16):
                gv = plsc.load_gather(wbuf, [gidx[g] + (boff + s)])
                new.append(accs[g] + _round_bf16(gv) * ws)
            return tuple(new)

        zero = jnp.zeros((16,), jnp.float32)
        accs = lax.fori_loop(0, SIZE, sbody, (zero, zero, zero, zero))

        work = list(accs)
        thresh = jnp.float32(0)
        for it in range(K):
            m01 = jnp.maximum(work[0], work[1])
            m23 = jnp.maximum(work[2], work[3])
            hm = jnp.max(jnp.maximum(m01, m23))
            if it == K - 1:
                thresh = hm
            else:
                work = [jnp.where(w == hm, NEG_INF, w) for w in work]
        for g in range(NPZ // 16):
            a = accs[g]
            obuf[pl.ds(k * NPZ + 16 * g, 16)] = jnp.where(
                a >= thresh, a, jnp.zeros((16,), jnp.float32))

    fire(0, 0, s0)

    @pl.loop(0, ZPW // 2)
    def _(k2):
        k = 2 * k2
        fire(k + 1, WZ, s1)
        wait(0, s0)
        compute_zone(k, 0)

        @pl.when(k + 2 < ZPW)
        def _():
            fire(k + 2, 0, s0)

        wait(WZ, s1)
        compute_zone(k + 1, WZ)

    cpo = pltpu.make_async_copy(
        obuf, o_hbm.at[pl.ds(z0 * NPZ, ZPW * NPZ)], so)
    cpo.start()
    cpo.wait()


def kernel(x, W):
    xf = jnp.reshape(x, (-1,))
    w2 = jnp.reshape(W, (NUM_ZONES, WZ))
    mesh = plsc.VectorSubcoreMesh(core_axis_name="c", subcore_axis_name="s")
    cp = pltpu.CompilerParams()
    if "needs_layout_passes" in pltpu.CompilerParams.__dataclass_fields__:
        cp = dataclasses.replace(cp, needs_layout_passes=False)
    sck = functools.partial(
        pl.kernel,
        mesh=mesh,
        compiler_params=cp,
        out_type=jax.ShapeDtypeStruct((NUM_ZONES * NPZ,), jnp.float32),
        scratch_types=[
            pltpu.VMEM((XSLAB,), jnp.float32),
            pltpu.VMEM((2 * WZ,), jnp.float32),
            pltpu.VMEM((ZPW * NPZ,), jnp.float32),
            pltpu.SemaphoreType.DMA,
            pltpu.SemaphoreType.DMA,
            pltpu.SemaphoreType.DMA,
            pltpu.SemaphoreType.DMA,
        ],
    )(_tec_body)
    out = sck(xf, w2)
    return jnp.reshape(out, (NUM_ZONES, NPZ))


# speedup vs baseline: 1.1766x; 1.1766x over previous
import dataclasses
import functools

import jax
import jax.numpy as jnp
from jax import lax
from jax.experimental import pallas as pl
from jax.experimental.pallas import tpu as pltpu
from jax.experimental.pallas import tpu_sc as plsc

INPUT_SIZE = 65536
SIZE = 128
STRIDE = 16
NPZ = 64
K = 8
NUM_ZONES = (INPUT_SIZE - (SIZE - 1)) // STRIDE

NW = 32
ZPW = 128
XSLAB = ZPW * STRIDE + SIZE
WZ = NPZ * SIZE
NEG_INF = float('-inf')


def _round_bf16(v):
    u = plsc.bitcast(v, jnp.int32)
    r = u + jnp.int32(0x7FFF) + ((u >> 16) & jnp.int32(1))
    return plsc.bitcast(r & jnp.int32(-65536), jnp.float32)


def _tec_body(x_hbm, w_hbm, o_hbm, xbuf, wbuf, obuf, sx, s0, s1, so):
    wid = lax.axis_index("s") * 2 + lax.axis_index("c")
    z0 = jnp.minimum(wid * ZPW, NUM_ZONES - ZPW)

    cpx = pltpu.make_async_copy(
        x_hbm.at[pl.ds(z0 * STRIDE, XSLAB)], xbuf, sx)
    cpx.start()
    cpx.wait()

    @pl.loop(0, XSLAB, step=16)
    def _(c):
        xbuf[pl.ds(c, 16)] = _round_bf16(xbuf[pl.ds(c, 16)])

    lane = lax.broadcasted_iota(jnp.int32, (16,), 0)
    gidx = [lane * SIZE + 16 * g * SIZE for g in range(NPZ // 16)]

    def fire(k, boff, sem):
        cp = pltpu.make_async_copy(
            w_hbm.at[z0 + k], wbuf.at[pl.ds(boff, WZ)], sem)
        cp.start()

    def wait(boff, sem):
        pltpu.make_async_copy(
            w_hbm.at[0], wbuf.at[pl.ds(boff, WZ)], sem).wait()

    UNROLL = 8

    def compute_zone(k, boff):
        xvec = jnp.full((16,), k * STRIDE, jnp.int32)

        def sbody(j, accs):
            s0 = j * UNROLL
            new = list(accs)
            for u in range(UNROLL):
                s = s0 + u
                ws = plsc.load_gather(xbuf, [xvec + s])
                for g in range(NPZ // 16):
                    gv = plsc.load_gather(wbuf, [gidx[g] + (boff + s)])
                    h = (u & 1) * (NPZ // 16)
                    new[g + h] = new[g + h] + _round_bf16(gv) * ws
            return tuple(new)

        zero = jnp.zeros((16,), jnp.float32)
        parts = lax.fori_loop(0, SIZE // UNROLL, sbody, (zero,) * (NPZ // 8))
        accs = [parts[g] + parts[g + NPZ // 16] for g in range(NPZ // 16)]

        work = list(accs)
        thresh = jnp.float32(0)
        for it in range(K):
            m01 = jnp.maximum(work[0], work[1])
            m23 = jnp.maximum(work[2], work[3])
            hm = jnp.max(jnp.maximum(m01, m23))
            if it == K - 1:
                thresh = hm
            else:
                work = [jnp.where(w == hm, NEG_INF, w) for w in work]
        for g in range(NPZ // 16):
            a = accs[g]
            obuf[pl.ds(k * NPZ + 16 * g, 16)] = jnp.where(
                a >= thresh, a, jnp.zeros((16,), jnp.float32))

    fire(0, 0, s0)

    @pl.loop(0, ZPW // 2)
    def _(k2):
        k = 2 * k2
        fire(k + 1, WZ, s1)
        wait(0, s0)
        compute_zone(k, 0)

        @pl.when(k + 2 < ZPW)
        def _():
            fire(k + 2, 0, s0)

        wait(WZ, s1)
        compute_zone(k + 1, WZ)

    cpo = pltpu.make_async_copy(
        obuf, o_hbm.at[pl.ds(z0 * NPZ, ZPW * NPZ)], so)
    cpo.start()
    cpo.wait()


def kernel(x, W):
    xf = jnp.reshape(x, (-1,))
    w2 = jnp.reshape(W, (NUM_ZONES, WZ))
    mesh = plsc.VectorSubcoreMesh(core_axis_name="c", subcore_axis_name="s")
    cp = pltpu.CompilerParams()
    if "needs_layout_passes" in pltpu.CompilerParams.__dataclass_fields__:
        cp = dataclasses.replace(cp, needs_layout_passes=False)
    sck = functools.partial(
        pl.kernel,
        mesh=mesh,
        compiler_params=cp,
        out_type=jax.ShapeDtypeStruct((NUM_ZONES * NPZ,), jnp.float32),
        scratch_types=[
            pltpu.VMEM((XSLAB,), jnp.float32),
            pltpu.VMEM((2 * WZ,), jnp.float32),
            pltpu.VMEM((ZPW * NPZ,), jnp.float32),
            pltpu.SemaphoreType.DMA,
            pltpu.SemaphoreType.DMA,
            pltpu.SemaphoreType.DMA,
            pltpu.SemaphoreType.DMA,
        ],
    )(_tec_body)
    out = sck(xf, w2)
    return jnp.reshape(out, (NUM_ZONES, NPZ))


# speedup vs baseline: 1.1767x; 1.0001x over previous
import dataclasses
import functools

import jax
import jax.numpy as jnp
from jax import lax
from jax.experimental import pallas as pl
from jax.experimental.pallas import tpu as pltpu
from jax.experimental.pallas import tpu_sc as plsc

INPUT_SIZE = 65536
SIZE = 128
STRIDE = 16
NPZ = 64
K = 8
NUM_ZONES = (INPUT_SIZE - (SIZE - 1)) // STRIDE

NW = 32
ZPW = 128
XSLAB = ZPW * STRIDE + SIZE
WZ = NPZ * SIZE
NEG_INF = float('-inf')


def _round_bf16(v):
    u = plsc.bitcast(v, jnp.int32)
    r = u + jnp.int32(0x7FFF) + ((u >> 16) & jnp.int32(1))
    return plsc.bitcast(r & jnp.int32(-65536), jnp.float32)


def _tec_body(x_hbm, w_hbm, o_hbm, xbuf, wbuf, obuf, sx, s0, s1, so):
    wid = lax.axis_index("s") * 2 + lax.axis_index("c")
    z0 = jnp.minimum(wid * ZPW, NUM_ZONES - ZPW)

    cpx = pltpu.make_async_copy(
        x_hbm.at[pl.ds(z0 * STRIDE, XSLAB)], xbuf, sx)
    cpx.start()
    cpx.wait()

    @pl.loop(0, XSLAB, step=16)
    def _(c):
        xbuf[pl.ds(c, 16)] = _round_bf16(xbuf[pl.ds(c, 16)])

    lane = lax.broadcasted_iota(jnp.int32, (16,), 0)
    gidx = [lane * SIZE + 16 * g * SIZE for g in range(NPZ // 16)]

    def fire(k, boff, sem):
        cp = pltpu.make_async_copy(
            w_hbm.at[z0 + k], wbuf.at[pl.ds(boff, WZ)], sem)
        cp.start()

    def wait(boff, sem):
        pltpu.make_async_copy(
            w_hbm.at[0], wbuf.at[pl.ds(boff, WZ)], sem).wait()

    UNROLL = 8

    def compute_zone(k, boff):
        xvec = jnp.full((16,), k * STRIDE, jnp.int32)

        def sbody(j, accs):
            s0 = j * UNROLL
            new = list(accs)
            for u in range(UNROLL):
                s = s0 + u
                ws = plsc.load_gather(xbuf, [xvec + s])
                for g in range(NPZ // 16):
                    gv = plsc.load_gather(wbuf, [gidx[g] + (boff + s)])
                    h = (u & 1) * (NPZ // 16)
                    new[g + h] = new[g + h] + _round_bf16(gv) * ws
            return tuple(new)

        zero = jnp.zeros((16,), jnp.float32)
        parts = plsc.parallel_loop(
            0, SIZE // UNROLL, carry=(zero,) * (NPZ // 8))(sbody)
        accs = [parts[g] + parts[g + NPZ // 16] for g in range(NPZ // 16)]

        work = list(accs)
        thresh = jnp.float32(0)
        for it in range(K):
            m01 = jnp.maximum(work[0], work[1])
            m23 = jnp.maximum(work[2], work[3])
            hm = jnp.max(jnp.maximum(m01, m23))
            if it == K - 1:
                thresh = hm
            else:
                work = [jnp.where(w == hm, NEG_INF, w) for w in work]
        for g in range(NPZ // 16):
            a = accs[g]
            obuf[pl.ds(k * NPZ + 16 * g, 16)] = jnp.where(
                a >= thresh, a, jnp.zeros((16,), jnp.float32))

    fire(0, 0, s0)

    @pl.loop(0, ZPW // 2)
    def _(k2):
        k = 2 * k2
        fire(k + 1, WZ, s1)
        wait(0, s0)
        compute_zone(k, 0)

        @pl.when(k + 2 < ZPW)
        def _():
            fire(k + 2, 0, s0)

        wait(WZ, s1)
        compute_zone(k + 1, WZ)

    cpo = pltpu.make_async_copy(
        obuf, o_hbm.at[pl.ds(z0 * NPZ, ZPW * NPZ)], so)
    cpo.start()
    cpo.wait()


def kernel(x, W):
    xf = jnp.reshape(x, (-1,))
    w2 = jnp.reshape(W, (NUM_ZONES, WZ))
    mesh = plsc.VectorSubcoreMesh(core_axis_name="c", subcore_axis_name="s")
    cp = pltpu.CompilerParams()
    if "needs_layout_passes" in pltpu.CompilerParams.__dataclass_fields__:
        cp = dataclasses.replace(cp, needs_layout_passes=False)
    sck = functools.partial(
        pl.kernel,
        mesh=mesh,
        compiler_params=cp,
        out_type=jax.ShapeDtypeStruct((NUM_ZONES * NPZ,), jnp.float32),
        scratch_types=[
            pltpu.VMEM((XSLAB,), jnp.float32),
            pltpu.VMEM((2 * WZ,), jnp.float32),
            pltpu.VMEM((ZPW * NPZ,), jnp.float32),
            pltpu.SemaphoreType.DMA,
            pltpu.SemaphoreType.DMA,
            pltpu.SemaphoreType.DMA,
            pltpu.SemaphoreType.DMA,
        ],
    )(_tec_body)
    out = sck(xf, w2)
    return jnp.reshape(out, (NUM_ZONES, NPZ))


# speedup vs baseline: 4.4023x; 3.7412x over previous
import dataclasses
import functools

import jax
import jax.numpy as jnp
from jax import lax
from jax.experimental import pallas as pl
from jax.experimental.pallas import tpu as pltpu
from jax.experimental.pallas import tpu_sc as plsc

INPUT_SIZE = 65536
SIZE = 128
STRIDE = 16
NPZ = 64
K = 8
NUM_ZONES = (INPUT_SIZE - (SIZE - 1)) // STRIDE

NEG_INF = float("-inf")


ZB = 128
TCZ = 3712
NB = TCZ // ZB
XROWS = INPUT_SIZE // SIZE


def _tc_body(xm_ref, xh_ref, w_ref, out_ref):
    xa = jnp.concatenate([xm_ref[...], xh_ref[...]], axis=0)
    b = jnp.roll(xa, -1, axis=0)
    lane = lax.broadcasted_iota(jnp.int32, (ZB // 8 + 8, SIZE), 1)
    rows = []
    for r in range(8):
        if r == 0:
            rr = xa
        else:
            rl = jnp.roll(xa, -16 * r, axis=1)
            rlb = jnp.roll(b, -16 * r, axis=1)
            rr = jnp.where(lane < SIZE - 16 * r, rl, rlb)
        rows.append(rr[:ZB // 8])
    win = jnp.stack(rows, axis=1).reshape(ZB, SIZE)
    win = win.astype(jnp.bfloat16)
    resp = lax.dot_general(w_ref[...].astype(jnp.bfloat16), win,
                           (((2,), (1,)), ((0,), (0,))),
                           preferred_element_type=jnp.float32)
    respT = resp.T
    work = respT
    for _ in range(K - 1):
        m = jnp.max(work, axis=0, keepdims=True)
        work = jnp.where(work == m, NEG_INF, work)
    thresh = jnp.max(work, axis=0, keepdims=True)
    outT = jnp.where(respT >= thresh, respT, jnp.zeros_like(respT))
    out_ref[...] = outT.T


def _tc_part(x, W):
    xs = jnp.reshape(x, (XROWS, SIZE))
    return pl.pallas_call(
        _tc_body,
        grid=(NB,),
        in_specs=[
            pl.BlockSpec((ZB // 8, SIZE), lambda i: (i, 0)),
            pl.BlockSpec((8, SIZE),
                         lambda i: (jnp.minimum((ZB // 64) * (i + 1),
                                                XROWS // 8 - 1), 0)),
            pl.BlockSpec((ZB, NPZ, SIZE), lambda i: (i, 0, 0)),
        ],
        out_specs=pl.BlockSpec((ZB, NPZ), lambda i: (i, 0)),
        out_shape=jax.ShapeDtypeStruct((TCZ, NPZ), jnp.float32),
    )(xs, xs, W)


NSC = NUM_ZONES - TCZ
NW = 32
ZPW = 12
XSLAB = ZPW * STRIDE + SIZE
WZ = NPZ * SIZE
UNROLL = 8


def _round_bf16(v):
    u = plsc.bitcast(v, jnp.int32)
    r = u + jnp.int32(0x7FFF) + ((u >> 16) & jnp.int32(1))
    return plsc.bitcast(r & jnp.int32(-65536), jnp.float32)


def _tec_body(x_hbm, w_hbm, o_hbm, xbuf, wbuf, obuf, sx, s0, s1, so):
    wid = lax.axis_index("s") * 2 + lax.axis_index("c")
    z0 = TCZ + jnp.minimum(wid * ZPW, NSC - ZPW)

    cpx = pltpu.make_async_copy(
        x_hbm.at[pl.ds(z0 * STRIDE, XSLAB)], xbuf, sx)
    cpx.start()
    cpx.wait()

    @pl.loop(0, XSLAB, step=16)
    def _(c):
        xbuf[pl.ds(c, 16)] = _round_bf16(xbuf[pl.ds(c, 16)])

    lane = lax.broadcasted_iota(jnp.int32, (16,), 0)
    gidx = [lane * SIZE + 16 * g * SIZE for g in range(NPZ // 16)]

    def fire(k, boff, sem):
        pltpu.make_async_copy(
            w_hbm.at[z0 + k], wbuf.at[pl.ds(boff, WZ)], sem).start()

    def wait(boff, sem):
        pltpu.make_async_copy(
            w_hbm.at[0], wbuf.at[pl.ds(boff, WZ)], sem).wait()

    def compute_zone(k, boff):
        xvec = jnp.full((16,), k * STRIDE, jnp.int32)

        def sbody(j, accs):
            s0_ = j * UNROLL
            new = list(accs)
            for u in range(UNROLL):
                s = s0_ + u
                ws = plsc.load_gather(xbuf, [xvec + s])
                for g in range(NPZ // 16):
                    gv = plsc.load_gather(wbuf, [gidx[g] + (boff + s)])
                    h = (u & 1) * (NPZ // 16)
                    new[g + h] = new[g + h] + _round_bf16(gv) * ws
            return tuple(new)

        zero = jnp.zeros((16,), jnp.float32)
        parts = plsc.parallel_loop(
            0, SIZE // UNROLL, carry=(zero,) * (NPZ // 8))(sbody)
        accs = [parts[g] + parts[g + NPZ // 16] for g in range(NPZ // 16)]

        work = list(accs)
        thresh = jnp.float32(0)
        for it in range(K):
            m01 = jnp.maximum(work[0], work[1])
            m23 = jnp.maximum(work[2], work[3])
            hm = jnp.max(jnp.maximum(m01, m23))
            if it == K - 1:
                thresh = hm
            else:
                work = [jnp.where(w == hm, NEG_INF, w) for w in work]
        for g in range(NPZ // 16):
            a = accs[g]
            obuf[pl.ds(k * NPZ + 16 * g, 16)] = jnp.where(
                a >= thresh, a, jnp.zeros((16,), jnp.float32))

    fire(0, 0, s0)

    @pl.loop(0, ZPW // 2)
    def _(k2):
        k = 2 * k2
        fire(k + 1, WZ, s1)
        wait(0, s0)
        compute_zone(k, 0)

        @pl.when(k + 2 < ZPW)
        def _():
            fire(k + 2, 0, s0)

        wait(WZ, s1)
        compute_zone(k + 1, WZ)

    cpo = pltpu.make_async_copy(
        obuf, o_hbm.at[pl.ds((z0 - TCZ) * NPZ, ZPW * NPZ)], so)
    cpo.start()
    cpo.wait()


def _sc_part(x, W):
    xf = jnp.reshape(x, (-1,))
    w2 = jnp.reshape(W, (NUM_ZONES, WZ))
    mesh = plsc.VectorSubcoreMesh(core_axis_name="c", subcore_axis_name="s")
    cp = pltpu.CompilerParams()
    if "needs_layout_passes" in pltpu.CompilerParams.__dataclass_fields__:
        cp = dataclasses.replace(cp, needs_layout_passes=False)
    sck = functools.partial(
        pl.kernel,
        mesh=mesh,
        compiler_params=cp,
        out_type=jax.ShapeDtypeStruct((NSC * NPZ,), jnp.float32),
        scratch_types=[
            pltpu.VMEM((XSLAB,), jnp.float32),
            pltpu.VMEM((2 * WZ,), jnp.float32),
            pltpu.VMEM((ZPW * NPZ,), jnp.float32),
            pltpu.SemaphoreType.DMA,
            pltpu.SemaphoreType.DMA,
            pltpu.SemaphoreType.DMA,
            pltpu.SemaphoreType.DMA,
        ],
    )(_tec_body)
    out = sck(xf, w2)
    return jnp.reshape(out, (NSC, NPZ))


def kernel(x, W):
    out_tc = _tc_part(x, W)
    out_sc = _sc_part(x, W)
    return jnp.concatenate([out_tc, out_sc], axis=0)


# speedup vs baseline: 8.4662x; 1.9231x over previous
import dataclasses
import functools

import jax
import jax.numpy as jnp
from jax import lax
from jax.experimental import pallas as pl
from jax.experimental.pallas import tpu as pltpu
from jax.experimental.pallas import tpu_sc as plsc

INPUT_SIZE = 65536
SIZE = 128
STRIDE = 16
NPZ = 64
K = 8
NUM_ZONES = (INPUT_SIZE - (SIZE - 1)) // STRIDE

NEG_INF = float("-inf")


ZB = 128
TCZ = 3584
NB = TCZ // ZB
XROWS = INPUT_SIZE // SIZE


def _tc_body(xm_ref, xh_ref, w_ref, out_ref):
    xa = jnp.concatenate([xm_ref[...], xh_ref[...]], axis=0)
    b = jnp.roll(xa, -1, axis=0)
    lane = lax.broadcasted_iota(jnp.int32, (ZB // 8 + 8, SIZE), 1)
    rows = []
    for r in range(8):
        if r == 0:
            rr = xa
        else:
            rl = jnp.roll(xa, -16 * r, axis=1)
            rlb = jnp.roll(b, -16 * r, axis=1)
            rr = jnp.where(lane < SIZE - 16 * r, rl, rlb)
        rows.append(rr[:ZB // 8])
    win = jnp.stack(rows, axis=1).reshape(ZB, SIZE)
    win = win.astype(jnp.bfloat16)
    resp = lax.dot_general(w_ref[...].astype(jnp.bfloat16), win,
                           (((2,), (1,)), ((0,), (0,))),
                           preferred_element_type=jnp.float32)
    respT = resp.T
    work = respT
    for _ in range(K - 1):
        m = jnp.max(work, axis=0, keepdims=True)
        work = jnp.where(work == m, NEG_INF, work)
    thresh = jnp.max(work, axis=0, keepdims=True)
    outT = jnp.where(respT >= thresh, respT, jnp.zeros_like(respT))
    out_ref[...] = outT.T


def _tc_part(x, W):
    xs = jnp.reshape(x, (XROWS, SIZE))
    return pl.pallas_call(
        _tc_body,
        grid=(NB,),
        in_specs=[
            pl.BlockSpec((ZB // 8, SIZE), lambda i: (i, 0)),
            pl.BlockSpec((8, SIZE),
                         lambda i: (jnp.minimum((ZB // 64) * (i + 1),
                                                XROWS // 8 - 1), 0)),
            pl.BlockSpec((ZB, NPZ, SIZE), lambda i: (i, 0, 0)),
        ],
        out_specs=pl.BlockSpec((ZB, NPZ), lambda i: (i, 0)),
        out_shape=jax.ShapeDtypeStruct((TCZ, NPZ), jnp.float32),
    )(xs, xs, W)


NSC = NUM_ZONES - TCZ
NW = 32
ZPW = 16
XSLAB = ZPW * STRIDE + SIZE
WZ = NPZ * SIZE
UNROLL = 8


def _round_bf16(v):
    u = plsc.bitcast(v, jnp.int32)
    r = u + jnp.int32(0x7FFF) + ((u >> 16) & jnp.int32(1))
    return plsc.bitcast(r & jnp.int32(-65536), jnp.float32)


def _tec_body(x_hbm, w_hbm, o_hbm, xbuf, wbuf, obuf, sx, s0, s1, so):
    wid = lax.axis_index("s") * 2 + lax.axis_index("c")
    z0 = TCZ + jnp.minimum(wid * ZPW, NSC - ZPW)

    cpx = pltpu.make_async_copy(
        x_hbm.at[pl.ds(z0 * STRIDE, XSLAB)], xbuf, sx)
    cpx.start()
    cpx.wait()

    @pl.loop(0, XSLAB, step=16)
    def _(c):
        xbuf[pl.ds(c, 16)] = _round_bf16(xbuf[pl.ds(c, 16)])

    lane = lax.broadcasted_iota(jnp.int32, (16,), 0)
    gidx = [lane + 16 * g for g in range(NPZ // 16)]

    def fire(k, b, sem):
        pltpu.make_async_copy(
            w_hbm.at[z0 + k], wbuf.at[b], sem).start()

    def wait(b, sem):
        pltpu.make_async_copy(
            w_hbm.at[0], wbuf.at[b], sem).wait()

    def compute_zone(k, b):
        xvec = jnp.full((16,), k * STRIDE, jnp.int32)
        bvec = jnp.full((16,), b, jnp.int32)

        def sbody(j, accs):
            s0_ = j * UNROLL
            new = list(accs)
            for u in range(UNROLL):
                s = s0_ + u
                ws = plsc.load_gather(xbuf, [xvec + s])
                svec = jnp.full((16,), s, jnp.int32)
                for g in range(NPZ // 16):
                    gv = plsc.load_gather(wbuf, [bvec, gidx[g], svec])
                    h = (u & 1) * (NPZ // 16)
                    new[g + h] = new[g + h] + _round_bf16(gv) * ws
            return tuple(new)

        zero = jnp.zeros((16,), jnp.float32)
        parts = plsc.parallel_loop(
            0, SIZE // UNROLL, carry=(zero,) * (NPZ // 8))(sbody)
        accs = [parts[g] + parts[g + NPZ // 16] for g in range(NPZ // 16)]

        work = list(accs)
        thresh = jnp.float32(0)
        for it in range(K):
            m01 = jnp.maximum(work[0], work[1])
            m23 = jnp.maximum(work[2], work[3])
            hm = jnp.max(jnp.maximum(m01, m23))
            if it == K - 1:
                thresh = hm
            else:
                work = [jnp.where(w == hm, NEG_INF, w) for w in work]
        for g in range(NPZ // 16):
            a = accs[g]
            obuf[k, pl.ds(16 * g, 16)] = jnp.where(
                a >= thresh, a, jnp.zeros((16,), jnp.float32))

    fire(0, 0, s0)

    @pl.loop(0, ZPW // 2)
    def _(k2):
        k = 2 * k2
        fire(k + 1, 1, s1)
        wait(0, s0)
        compute_zone(k, 0)

        @pl.when(k + 2 < ZPW)
        def _():
            fire(k + 2, 0, s0)

        wait(1, s1)
        compute_zone(k + 1, 1)

    cpo = pltpu.make_async_copy(
        obuf, o_hbm.at[pl.ds(z0 - TCZ, ZPW)], so)
    cpo.start()
    cpo.wait()


def _sc_part(x, W):
    xf = jnp.reshape(x, (-1,))
    mesh = plsc.VectorSubcoreMesh(core_axis_name="c", subcore_axis_name="s")
    cp = pltpu.CompilerParams()
    if "needs_layout_passes" in pltpu.CompilerParams.__dataclass_fields__:
        cp = dataclasses.replace(cp, needs_layout_passes=False)
    sck = functools.partial(
        pl.kernel,
        mesh=mesh,
        compiler_params=cp,
        out_type=jax.ShapeDtypeStruct((NSC, NPZ), jnp.float32),
        scratch_types=[
            pltpu.VMEM((XSLAB,), jnp.float32),
            pltpu.VMEM((2, NPZ, SIZE), jnp.float32),
            pltpu.VMEM((ZPW, NPZ), jnp.float32),
            pltpu.SemaphoreType.DMA,
            pltpu.SemaphoreType.DMA,
            pltpu.SemaphoreType.DMA,
            pltpu.SemaphoreType.DMA,
        ],
    )(_tec_body)
    return sck(xf, W)


def kernel(x, W):
    out_tc = _tc_part(x, W)
    out_sc = _sc_part(x, W)
    return jnp.concatenate([out_tc, out_sc], axis=0)
